# baseline (speedup 1.0000x reference)
import jax
import jax.numpy as jnp
from jax.experimental import pallas as pl
from jax.experimental.pallas import tpu as pltpu

ROW_BLK = 256
COL_TILE = 2048
NCODES = 8192
BIG = 2 ** 30


def _bf16(v):
    return v.astype(jnp.bfloat16).astype(jnp.float32)


def _dist_argmin_body(flat_ref, cbt_ref, xs_ref, cs_ref, idx_ref, val_ref):
    xs = xs_ref[...]
    lane = jax.lax.broadcasted_iota(jnp.int32, (ROW_BLK, 128), 1)
    acc_v = None
    acc_i = None
    acc_e = None

    for j in range(NCODES // COL_TILE):
        c0 = j * COL_TILE
        t = jax.lax.dot_general(
            flat_ref[...], cbt_ref[:, c0:c0 + COL_TILE],
            (((1,), (0,)), ((), ())), preferred_element_type=jnp.float32)
        rmin = None
        rgrp = None
        for c in range(COL_TILE // 128):
            d = (xs + cs_ref[:, c0 + c * 128:c0 + (c + 1) * 128]) \
                - 2.0 * t[:, c * 128:(c + 1) * 128]
            if rmin is None:
                rmin = d
                rgrp = jnp.zeros((ROW_BLK, 128), jnp.int32)
            else:
                upd = d < rmin
                rmin = jnp.where(upd, d, rmin)
                rgrp = jnp.where(upd, c, rgrp)
        gidx = rgrp * 128 + lane + c0
        pmin = jnp.min(rmin, axis=1, keepdims=True)
        pidx = jnp.where(rmin == pmin, gidx, jnp.int32(BIG))
        parg = jnp.min(pidx, axis=1, keepdims=True)

        if acc_v is None:
            acc_v, acc_i, acc_e = _bf16(pmin), parg, pmin
        else:
            upd = pmin < acc_v
            acc_v = _bf16(jnp.where(upd, pmin, acc_v))
            acc_i = jnp.where(upd, parg, acc_i)
            acc_e = jnp.where(upd, pmin, acc_e)

    idx_ref[...] = acc_i
    val_ref[...] = acc_e


def _dist_argmin(flat, cbt, x_sqr, cb_sqr):
    n, d = flat.shape
    k = cbt.shape[1]
    return pl.pallas_call(
        _dist_argmin_body,
        grid=(n // ROW_BLK,),
        in_specs=[
            pl.BlockSpec((ROW_BLK, d), lambda i: (i, 0)),
            pl.BlockSpec((d, k), lambda i: (0, 0)),
            pl.BlockSpec((ROW_BLK, 1), lambda i: (i, 0)),
            pl.BlockSpec((1, k), lambda i: (0, 0)),
        ],
        out_specs=[
            pl.BlockSpec((ROW_BLK, 1), lambda i: (i, 0)),
            pl.BlockSpec((ROW_BLK, 1), lambda i: (i, 0)),
        ],
        out_shape=[
            jax.ShapeDtypeStruct((n, 1), jnp.int32),
            jax.ShapeDtypeStruct((n, 1), jnp.float32),
        ],
        compiler_params=pltpu.CompilerParams(
            dimension_semantics=("parallel",)),
    )(flat, cbt, x_sqr, cb_sqr)


def kernel(x, codebook):
    d = x.shape[-1]
    flat = x.reshape(-1, d)
    x_sqr = jnp.sum(flat ** 2, axis=1, keepdims=True)
    cb_sqr = jnp.sum(codebook ** 2, axis=1)[None, :]
    best_idx, best_val = _dist_argmin(flat, codebook.T, x_sqr, cb_sqr)
    idx_flat = best_idx[:, 0]
    zq = jnp.take(codebook, idx_flat, axis=0)
    qe = (flat + (zq - flat)).reshape(x.shape)
    commit_loss = jnp.mean(best_val) / d
    indices = idx_flat.reshape(x.shape[:-1])
    return qe, commit_loss, indices


# speedup vs baseline: 1.0639x; 1.0639x over previous
import jax
import jax.numpy as jnp
from jax.experimental import pallas as pl
from jax.experimental.pallas import tpu as pltpu

ROW_BLK = 1024
COL_TILE = 2048
NCODES = 8192
BIG = 2 ** 30


def _bf16(v):
    return v.astype(jnp.bfloat16).astype(jnp.float32)


def _dist_argmin_body(flat_ref, cb_ref, xs_ref, cs_ref, idx_ref, val_ref):
    xs = xs_ref[...]
    acc_v = None
    acc_i = None
    acc_e = None

    for j in range(NCODES // COL_TILE):
        c0 = j * COL_TILE
        t = jax.lax.dot_general(
            flat_ref[...], cb_ref[c0:c0 + COL_TILE, :],
            (((1,), (1,)), ((), ())), preferred_element_type=jnp.float32)
        dist = (xs + cs_ref[:, c0:c0 + COL_TILE]) - 2.0 * t
        pmin = jnp.min(dist, axis=1, keepdims=True)
        gcol = jax.lax.broadcasted_iota(jnp.int32, dist.shape, 1) + c0
        pidx = jnp.where(dist == pmin, gcol, jnp.int32(BIG))
        parg = jnp.min(pidx, axis=1, keepdims=True)
        if acc_v is None:
            acc_v, acc_i, acc_e = _bf16(pmin), parg, pmin
        else:
            upd = pmin < acc_v
            acc_v = _bf16(jnp.where(upd, pmin, acc_v))
            acc_i = jnp.where(upd, parg, acc_i)
            acc_e = jnp.where(upd, pmin, acc_e)

    idx_ref[...] = acc_i
    val_ref[...] = acc_e


def _dist_argmin(flat, codebook, x_sqr, cb_sqr):
    n, d = flat.shape
    k = codebook.shape[0]
    return pl.pallas_call(
        _dist_argmin_body,
        grid=(n // ROW_BLK,),
        in_specs=[
            pl.BlockSpec((ROW_BLK, d), lambda i: (i, 0)),
            pl.BlockSpec((k, d), lambda i: (0, 0)),
            pl.BlockSpec((ROW_BLK, 1), lambda i: (i, 0)),
            pl.BlockSpec((1, k), lambda i: (0, 0)),
        ],
        out_specs=[
            pl.BlockSpec((ROW_BLK, 1), lambda i: (i, 0)),
            pl.BlockSpec((ROW_BLK, 1), lambda i: (i, 0)),
        ],
        out_shape=[
            jax.ShapeDtypeStruct((n, 1), jnp.int32),
            jax.ShapeDtypeStruct((n, 1), jnp.float32),
        ],
        compiler_params=pltpu.CompilerParams(
            dimension_semantics=("parallel",)),
    )(flat, codebook, x_sqr, cb_sqr)


def kernel(x, codebook):
    d = x.shape[-1]
    flat = x.reshape(-1, d)
    x_sqr = jnp.sum(flat ** 2, axis=1, keepdims=True)
    cb_sqr = jnp.sum(codebook ** 2, axis=1)[None, :]
    best_idx, best_val = _dist_argmin(flat, codebook, x_sqr, cb_sqr)
    idx_flat = best_idx[:, 0]
    zq = jnp.take(codebook, idx_flat, axis=0)
    qe = (flat + (zq - flat)).reshape(x.shape)
    commit_loss = jnp.mean(best_val) / d
    indices = idx_flat.reshape(x.shape[:-1])
    return qe, commit_loss, indices
